# Initial kernel scaffold; baseline (speedup 1.0000x reference)
import jax
import jax.numpy as jnp
from jax.experimental import pallas as pl


def kernel(x, pos, batch, W, b, gamma, beta):
    raise NotImplementedError("write your pallas kernel here")


# speedup vs baseline: 48.8304x; 48.8304x over previous
import functools

import jax
import jax.numpy as jnp
from jax.experimental import pallas as pl
from jax.experimental.pallas import tpu as pltpu

_R2 = 0.25
_KMAX = 128
_LN_EPS = 1e-5
_BITER = 30
_BR = 128
_BLK_PREP = 512


def _prep_body(x_ref, p8_ref, w1_ref, w38_ref, b_ref,
               ac_ref, va_ref, bc_ref, vb_ref):
    d = x_ref.shape[1]
    bm = jnp.dot(p8_ref[:], w38_ref[:], preferred_element_type=jnp.float32)
    a = jnp.dot(x_ref[:], w1_ref[:], preferred_element_type=jnp.float32)
    a = a + bm + b_ref[:]
    mu = jnp.mean(a, axis=1, keepdims=True)
    acv = a - mu
    ac_ref[:] = acv
    va_ref[:] = jnp.mean(acv * acv, axis=1, keepdims=True)
    mub = jnp.mean(bm, axis=1, keepdims=True)
    bcv = bm - mub
    bc_ref[:] = bcv
    vb_ref[:] = jnp.mean(bcv * bcv, axis=1, keepdims=True)


def _main_body(n_real, br, npad, d,
               posT_ref, pblk_ref, ac_ref, acT_ref, va_ref, bc_ref, vb_ref,
               gam_ref, bet_ref, out_ref, d2_ref, w_ref):
    i = pl.program_id(0)
    posT = posT_ref[:]
    sq_row = jnp.sum(posT * posT, axis=0, keepdims=True)
    pb = pblk_ref[:]
    sq_blk = jnp.sum(pb * pb, axis=1, keepdims=True)
    d2 = sq_blk + sq_row - 2.0 * jnp.dot(
        pb, posT, preferred_element_type=jnp.float32)
    col = jax.lax.broadcasted_iota(jnp.int32, (br, npad), 1)
    row = jax.lax.broadcasted_iota(jnp.int32, (br, npad), 0) + i * br
    isdiag = col == row
    validc = (col < n_real) & jnp.logical_not(isdiag)
    d2_ref[:] = jnp.where(validc, d2, jnp.float32(jnp.inf))

    kf = jnp.float32(_KMAX)

    def bs_body(_, carry):
        lo, hi = carry
        mid = 0.5 * (lo + hi)
        cnt = jnp.sum((d2_ref[:] <= mid).astype(jnp.float32),
                      axis=1, keepdims=True)
        ge = cnt >= kf
        return (jnp.where(ge, lo, mid), jnp.where(ge, mid, hi))

    lo0 = jnp.full((br, 1), -1e-3, jnp.float32)
    hi0 = jnp.full((br, 1), _R2, jnp.float32)
    _, hi = jax.lax.fori_loop(0, _BITER, bs_body, (lo0, hi0))

    mask = (d2_ref[:] <= hi) | (isdiag & (col < n_real))
    cnt = jnp.sum(mask.astype(jnp.float32), axis=1, keepdims=True)

    w_ref[:] = jnp.dot(bc_ref[:], acT_ref[:],
                       preferred_element_type=jnp.float32)
    var = va_ref[:] + vb_ref[:] - (2.0 / d) * w_ref[:]
    w_ref[:] = jnp.where(mask, jax.lax.rsqrt(var + _LN_EPS), 0.0)
    s0 = jnp.sum(w_ref[:], axis=1, keepdims=True)
    s1 = jnp.dot(w_ref[:], ac_ref[:],
                 preferred_element_type=jnp.float32)
    o = (s1 - s0 * bc_ref[:]) * (gam_ref[:] / jnp.maximum(cnt, 1.0)) + bet_ref[:]
    out_ref[:] = jnp.maximum(o, 0.0)


def kernel(x, pos, batch, W, b, gamma, beta):
    n, d = x.shape
    npad = ((n + _BLK_PREP - 1) // _BLK_PREP) * _BLK_PREP
    if npad % _BR:
        npad = ((npad + _BR - 1) // _BR) * _BR
    xp = jnp.pad(x, ((0, npad - n), (0, 0)))
    p8 = jnp.pad(pos, ((0, npad - n), (0, 5)))
    w1 = W[:d]
    w38 = jnp.pad(W[d:], ((0, 5), (0, 0)))
    b_row = b.reshape(1, d)
    gam = gamma.reshape(1, d)
    bet = beta.reshape(1, d)

    ac, va, bc, vb = pl.pallas_call(
        _prep_body,
        grid=(npad // _BLK_PREP,),
        in_specs=[
            pl.BlockSpec((_BLK_PREP, d), lambda i: (i, 0)),
            pl.BlockSpec((_BLK_PREP, 8), lambda i: (i, 0)),
            pl.BlockSpec((d, d), lambda i: (0, 0)),
            pl.BlockSpec((8, d), lambda i: (0, 0)),
            pl.BlockSpec((1, d), lambda i: (0, 0)),
        ],
        out_specs=[
            pl.BlockSpec((_BLK_PREP, d), lambda i: (i, 0)),
            pl.BlockSpec((_BLK_PREP, 1), lambda i: (i, 0)),
            pl.BlockSpec((_BLK_PREP, d), lambda i: (i, 0)),
            pl.BlockSpec((_BLK_PREP, 1), lambda i: (i, 0)),
        ],
        out_shape=[
            jax.ShapeDtypeStruct((npad, d), jnp.float32),
            jax.ShapeDtypeStruct((npad, 1), jnp.float32),
            jax.ShapeDtypeStruct((npad, d), jnp.float32),
            jax.ShapeDtypeStruct((npad, 1), jnp.float32),
        ],
    )(xp, p8, w1, w38, b_row)

    posT = p8.T
    acT = ac.T
    va_row = va.reshape(1, npad)
    vb_col = vb

    out = pl.pallas_call(
        functools.partial(_main_body, n, _BR, npad, d),
        grid=(npad // _BR,),
        in_specs=[
            pl.BlockSpec((8, npad), lambda i: (0, 0)),
            pl.BlockSpec((_BR, 8), lambda i: (i, 0)),
            pl.BlockSpec((npad, d), lambda i: (0, 0)),
            pl.BlockSpec((d, npad), lambda i: (0, 0)),
            pl.BlockSpec((1, npad), lambda i: (0, 0)),
            pl.BlockSpec((_BR, d), lambda i: (i, 0)),
            pl.BlockSpec((_BR, 1), lambda i: (i, 0)),
            pl.BlockSpec((1, d), lambda i: (0, 0)),
            pl.BlockSpec((1, d), lambda i: (0, 0)),
        ],
        out_specs=pl.BlockSpec((_BR, d), lambda i: (i, 0)),
        out_shape=jax.ShapeDtypeStruct((npad, d), jnp.float32),
        scratch_shapes=[
            pltpu.VMEM((_BR, npad), jnp.float32),
            pltpu.VMEM((_BR, npad), jnp.float32),
        ],
    )(posT, p8, ac, acT, va_row, bc, vb_col, gam, bet)

    return out[:n]


# speedup vs baseline: 95.9285x; 1.9645x over previous
import functools

import jax
import jax.numpy as jnp
from jax.experimental import pallas as pl
from jax.experimental.pallas import tpu as pltpu

_R2 = 0.25
_KMAX = 128
_LN_EPS = 1e-5
_BITER = 12
_BR = 128
_BLK_PREP = 512
_AUGC = 256


def _prep_body(x_ref, p8_ref, w1_ref, w38_ref, b_ref,
               acaug_ref, acbf_ref, va_ref, bc_ref, vb_ref):
    d = x_ref.shape[1]
    p8 = p8_ref[:]
    bm = jnp.dot(p8, w38_ref[:], preferred_element_type=jnp.float32)
    a = jnp.dot(x_ref[:], w1_ref[:], preferred_element_type=jnp.float32)
    a = a + bm + b_ref[:]
    mu = jnp.mean(a, axis=1, keepdims=True)
    acv = a - mu
    acb = acv.astype(jnp.bfloat16)
    acbf_ref[:] = acb
    acaug_ref[:] = jnp.concatenate(
        [acb, jnp.ones_like(acb[:, :1]),
         jnp.zeros_like(acb[:, : _AUGC - d - 1])], axis=1)
    va_ref[:] = jnp.mean(acv * acv, axis=1, keepdims=True)
    mub = jnp.mean(bm, axis=1, keepdims=True)
    bcv = bm - mub
    bc_ref[:] = bcv
    vb_ref[:] = jnp.mean(bcv * bcv, axis=1, keepdims=True)


def _main_body(n_real, br, npad, d,
               pb_ref, posT_ref, acaug_ref, acTbf_ref, va_ref, bc_ref,
               vb_ref, gam_ref, bet_ref, out_ref, d2_ref, g_ref, wbf_ref):
    i = pl.program_id(0)
    posT = posT_ref[:]
    sq_row = jnp.sum(posT * posT, axis=0, keepdims=True)
    pb = pb_ref[:]
    sq_blk = jnp.sum(pb * pb, axis=1, keepdims=True)
    d2v = sq_blk + sq_row - 2.0 * jnp.dot(
        pb, posT, preferred_element_type=jnp.float32)
    col = jax.lax.broadcasted_iota(jnp.int32, (br, npad), 1)
    row = jax.lax.broadcasted_iota(jnp.int32, (br, npad), 0) + i * br
    d2_ref[:] = jnp.where(col == row, jnp.float32(-1e9), d2v)

    kf = jnp.float32(_KMAX + 1)

    def bs_body(_, carry):
        lo, hi = carry
        mid = 0.5 * (lo + hi)
        cnt = jnp.sum((d2_ref[:] <= mid).astype(jnp.float32),
                      axis=1, keepdims=True)
        ge = cnt >= kf
        return (jnp.where(ge, lo, mid), jnp.where(ge, mid, hi))

    lo0 = jnp.full((br, 1), -1e-3, jnp.float32)
    hi0 = jnp.full((br, 1), _R2, jnp.float32)
    _, hi = jax.lax.fori_loop(0, _BITER, bs_body, (lo0, hi0))

    g_ref[:] = jnp.dot(bc_ref[:].astype(jnp.bfloat16), acTbf_ref[:],
                       preferred_element_type=jnp.float32)
    mask = d2_ref[:] <= hi
    cnt = jnp.sum(mask.astype(jnp.float32), axis=1, keepdims=True)
    var = va_ref[:] + vb_ref[:] - (2.0 / d) * g_ref[:]
    wbf_ref[:] = jnp.where(mask, jax.lax.rsqrt(var + _LN_EPS),
                           0.0).astype(jnp.bfloat16)
    s1a = jnp.dot(wbf_ref[:], acaug_ref[:],
                  preferred_element_type=jnp.float32)
    s1 = s1a[:, :d]
    s0 = s1a[:, d:d + 1]
    o = (s1 - s0 * bc_ref[:]) * (gam_ref[:] / jnp.maximum(cnt, 1.0)) + bet_ref[:]
    out_ref[:] = jnp.maximum(o, 0.0)


def kernel(x, pos, batch, W, b, gamma, beta):
    n, d = x.shape
    npad = ((n + _BLK_PREP - 1) // _BLK_PREP) * _BLK_PREP
    if npad % _BR:
        npad = ((npad + _BR - 1) // _BR) * _BR
    nex = npad - n
    xp = jnp.pad(x, ((0, nex), (0, 0)))
    far = 1000.0 + 100.0 * jnp.arange(nex, dtype=jnp.float32)
    p_pad = jnp.concatenate([pos, jnp.broadcast_to(far[:, None], (nex, 3))], 0)
    p8 = jnp.pad(p_pad, ((0, 0), (0, 5)))
    posT = p8.T
    w1 = W[:d]
    w38 = jnp.pad(W[d:], ((0, 5), (0, 0)))
    b_row = b.reshape(1, d)
    gam = gamma.reshape(1, d)
    bet = beta.reshape(1, d)

    acaug, acbf, va, bc, vb = pl.pallas_call(
        _prep_body,
        grid=(npad // _BLK_PREP,),
        in_specs=[
            pl.BlockSpec((_BLK_PREP, d), lambda i: (i, 0)),
            pl.BlockSpec((_BLK_PREP, 8), lambda i: (i, 0)),
            pl.BlockSpec((d, d), lambda i: (0, 0)),
            pl.BlockSpec((8, d), lambda i: (0, 0)),
            pl.BlockSpec((1, d), lambda i: (0, 0)),
        ],
        out_specs=[
            pl.BlockSpec((_BLK_PREP, _AUGC), lambda i: (i, 0)),
            pl.BlockSpec((_BLK_PREP, d), lambda i: (i, 0)),
            pl.BlockSpec((_BLK_PREP, 1), lambda i: (i, 0)),
            pl.BlockSpec((_BLK_PREP, d), lambda i: (i, 0)),
            pl.BlockSpec((_BLK_PREP, 1), lambda i: (i, 0)),
        ],
        out_shape=[
            jax.ShapeDtypeStruct((npad, _AUGC), jnp.bfloat16),
            jax.ShapeDtypeStruct((npad, d), jnp.bfloat16),
            jax.ShapeDtypeStruct((npad, 1), jnp.float32),
            jax.ShapeDtypeStruct((npad, d), jnp.float32),
            jax.ShapeDtypeStruct((npad, 1), jnp.float32),
        ],
    )(xp, p8, w1, w38, b_row)

    acTbf = acbf.T
    va_row = va.reshape(1, npad)

    out = pl.pallas_call(
        functools.partial(_main_body, n, _BR, npad, d),
        grid=(npad // _BR,),
        in_specs=[
            pl.BlockSpec((_BR, 8), lambda i: (i, 0)),
            pl.BlockSpec((8, npad), lambda i: (0, 0)),
            pl.BlockSpec((npad, _AUGC), lambda i: (0, 0)),
            pl.BlockSpec((d, npad), lambda i: (0, 0)),
            pl.BlockSpec((1, npad), lambda i: (0, 0)),
            pl.BlockSpec((_BR, d), lambda i: (i, 0)),
            pl.BlockSpec((_BR, 1), lambda i: (i, 0)),
            pl.BlockSpec((1, d), lambda i: (0, 0)),
            pl.BlockSpec((1, d), lambda i: (0, 0)),
        ],
        out_specs=pl.BlockSpec((_BR, d), lambda i: (i, 0)),
        out_shape=jax.ShapeDtypeStruct((npad, d), jnp.float32),
        scratch_shapes=[
            pltpu.VMEM((_BR, npad), jnp.float32),
            pltpu.VMEM((_BR, npad), jnp.float32),
            pltpu.VMEM((_BR, npad), jnp.bfloat16),
        ],
    )(p8, posT, acaug, acTbf, va_row, bc, vb, gam, bet)

    return out[:n]


# speedup vs baseline: 112.2963x; 1.1706x over previous
import functools

import jax
import jax.numpy as jnp
from jax.experimental import pallas as pl
from jax.experimental.pallas import tpu as pltpu

_R2 = 0.25
_KMAX = 128
_LN_EPS = 1e-5
_BITER = 14
_BR = 128
_BLK_PREP = 512
_AUGC = 256


def _prep_body(x_ref, p8_ref, w1_ref, w38_ref, b_ref,
               acaug_ref, acbf_ref, va_ref, bc_ref, vb_ref):
    d = x_ref.shape[1]
    p8 = p8_ref[:]
    bm = jnp.dot(p8, w38_ref[:], preferred_element_type=jnp.float32)
    a = jnp.dot(x_ref[:], w1_ref[:], preferred_element_type=jnp.float32)
    a = a + bm + b_ref[:]
    mu = jnp.mean(a, axis=1, keepdims=True)
    acv = a - mu
    acb = acv.astype(jnp.bfloat16)
    acbf_ref[:] = acb
    acaug_ref[:] = jnp.concatenate(
        [acb, jnp.ones_like(acb[:, :1]),
         jnp.zeros_like(acb[:, : _AUGC - d - 1])], axis=1)
    va_ref[:] = jnp.mean(acv * acv, axis=1, keepdims=True)
    mub = jnp.mean(bm, axis=1, keepdims=True)
    bcv = bm - mub
    bc_ref[:] = bcv
    vb_ref[:] = jnp.mean(bcv * bcv, axis=1, keepdims=True)


def _main_body(n_real, br, npad, d,
               pb_ref, posT_ref, acaug_ref, acTbf_ref, va_ref, bc_ref,
               vb_ref, gam_ref, bet_ref, out_ref, d2_ref, g_ref, wbf_ref,
               hi_ref):
    i = pl.program_id(0)
    posT = posT_ref[:]
    sq_row = jnp.sum(posT * posT, axis=0, keepdims=True)
    pb = pb_ref[:]
    sq_blk = jnp.sum(pb * pb, axis=1, keepdims=True)
    d2v = sq_blk + sq_row - 2.0 * jnp.dot(
        pb, posT, preferred_element_type=jnp.float32)
    col = jax.lax.broadcasted_iota(jnp.int32, (br, npad), 1)
    row = jax.lax.broadcasted_iota(jnp.int32, (br, npad), 0) + i * br
    d2_ref[:] = jnp.where(col == row, jnp.float32(-1e9), d2v)

    kf = jnp.float32(_KMAX + 1)

    cnt0 = jnp.sum((d2_ref[:] <= jnp.float32(_R2)).astype(jnp.float32),
                   axis=1, keepdims=True)
    hi_ref[:] = jnp.full((br, 1), _R2, jnp.float32)

    @pl.when(jnp.max(cnt0) >= kf)
    def _search():
        def bs_body(_, carry):
            lo, hi = carry
            mid = 0.5 * (lo + hi)
            cnt = jnp.sum((d2_ref[:] <= mid).astype(jnp.float32),
                          axis=1, keepdims=True)
            ge = cnt >= kf
            return (jnp.where(ge, lo, mid), jnp.where(ge, mid, hi))

        lo0 = jnp.full((br, 1), -1e-3, jnp.float32)
        hi0 = jnp.full((br, 1), _R2, jnp.float32)
        _, hi = jax.lax.fori_loop(0, _BITER, bs_body, (lo0, hi0))
        hi_ref[:] = hi

    hi = hi_ref[:]

    g_ref[:] = jnp.dot(bc_ref[:].astype(jnp.bfloat16), acTbf_ref[:],
                       preferred_element_type=jnp.float32)
    mask = d2_ref[:] <= hi
    cnt = jnp.sum(mask.astype(jnp.float32), axis=1, keepdims=True)
    var = va_ref[:] + vb_ref[:] - (2.0 / d) * g_ref[:]
    wbf_ref[:] = jnp.where(mask, jax.lax.rsqrt(var + _LN_EPS),
                           0.0).astype(jnp.bfloat16)
    s1a = jnp.dot(wbf_ref[:], acaug_ref[:],
                  preferred_element_type=jnp.float32)
    s1 = s1a[:, :d]
    s0 = s1a[:, d:d + 1]
    o = (s1 - s0 * bc_ref[:]) * (gam_ref[:] / jnp.maximum(cnt, 1.0)) + bet_ref[:]
    out_ref[:] = jnp.maximum(o, 0.0)


def kernel(x, pos, batch, W, b, gamma, beta):
    n, d = x.shape
    npad = ((n + _BLK_PREP - 1) // _BLK_PREP) * _BLK_PREP
    if npad % _BR:
        npad = ((npad + _BR - 1) // _BR) * _BR
    nex = npad - n
    perm = jnp.argsort(jnp.sum(pos * pos, axis=1))
    inv = jnp.argsort(perm)
    x = x[perm]
    pos = pos[perm]
    xp = jnp.pad(x, ((0, nex), (0, 0)))
    far = 1000.0 + 100.0 * jnp.arange(nex, dtype=jnp.float32)
    p_pad = jnp.concatenate([pos, jnp.broadcast_to(far[:, None], (nex, 3))], 0)
    p8 = jnp.pad(p_pad, ((0, 0), (0, 5)))
    posT = p8.T
    w1 = W[:d]
    w38 = jnp.pad(W[d:], ((0, 5), (0, 0)))
    b_row = b.reshape(1, d)
    gam = gamma.reshape(1, d)
    bet = beta.reshape(1, d)

    acaug, acbf, va, bc, vb = pl.pallas_call(
        _prep_body,
        grid=(npad // _BLK_PREP,),
        in_specs=[
            pl.BlockSpec((_BLK_PREP, d), lambda i: (i, 0)),
            pl.BlockSpec((_BLK_PREP, 8), lambda i: (i, 0)),
            pl.BlockSpec((d, d), lambda i: (0, 0)),
            pl.BlockSpec((8, d), lambda i: (0, 0)),
            pl.BlockSpec((1, d), lambda i: (0, 0)),
        ],
        out_specs=[
            pl.BlockSpec((_BLK_PREP, _AUGC), lambda i: (i, 0)),
            pl.BlockSpec((_BLK_PREP, d), lambda i: (i, 0)),
            pl.BlockSpec((_BLK_PREP, 1), lambda i: (i, 0)),
            pl.BlockSpec((_BLK_PREP, d), lambda i: (i, 0)),
            pl.BlockSpec((_BLK_PREP, 1), lambda i: (i, 0)),
        ],
        out_shape=[
            jax.ShapeDtypeStruct((npad, _AUGC), jnp.bfloat16),
            jax.ShapeDtypeStruct((npad, d), jnp.bfloat16),
            jax.ShapeDtypeStruct((npad, 1), jnp.float32),
            jax.ShapeDtypeStruct((npad, d), jnp.float32),
            jax.ShapeDtypeStruct((npad, 1), jnp.float32),
        ],
    )(xp, p8, w1, w38, b_row)

    acTbf = acbf.T
    va_row = va.reshape(1, npad)

    out = pl.pallas_call(
        functools.partial(_main_body, n, _BR, npad, d),
        grid=(npad // _BR,),
        in_specs=[
            pl.BlockSpec((_BR, 8), lambda i: (i, 0)),
            pl.BlockSpec((8, npad), lambda i: (0, 0)),
            pl.BlockSpec((npad, _AUGC), lambda i: (0, 0)),
            pl.BlockSpec((d, npad), lambda i: (0, 0)),
            pl.BlockSpec((1, npad), lambda i: (0, 0)),
            pl.BlockSpec((_BR, d), lambda i: (i, 0)),
            pl.BlockSpec((_BR, 1), lambda i: (i, 0)),
            pl.BlockSpec((1, d), lambda i: (0, 0)),
            pl.BlockSpec((1, d), lambda i: (0, 0)),
        ],
        out_specs=pl.BlockSpec((_BR, d), lambda i: (i, 0)),
        out_shape=jax.ShapeDtypeStruct((npad, d), jnp.float32),
        scratch_shapes=[
            pltpu.VMEM((_BR, npad), jnp.float32),
            pltpu.VMEM((_BR, npad), jnp.float32),
            pltpu.VMEM((_BR, npad), jnp.bfloat16),
            pltpu.VMEM((_BR, 1), jnp.float32),
        ],
    )(p8, posT, acaug, acTbf, va_row, bc, vb, gam, bet)

    return out[:n][inv]
